# baseline (speedup 1.0000x reference)
import jax
import jax.numpy as jnp
from jax import lax
from jax.experimental import pallas as pl
from jax.experimental.pallas import tpu as pltpu
from jax.experimental.pallas import tpu_sc as plsc

_D = 64
_NC = 2
_NS = 16
_NW = _NC * _NS
_L = 16
_BLK = 128


def _gather_body(idx_hbm, table_hbm, out_hbm, idx_v, ridx, rbuf, tbuf,
                 g0, g1, w0, w1):
    gsem = (g0, g1)
    wsem = (w0, w1)
    wid = lax.axis_index("s") * _NC + lax.axis_index("c")
    hist = idx_hbm.shape[0]
    b0 = wid * _BLK
    pltpu.sync_copy(idx_hbm.at[:, pl.ds(b0, _BLK)], idx_v)

    lane = lax.iota(jnp.int32, _L)
    lvecs = [lane + lg * _L for lg in range(_BLK // _L)]

    def prep_and_fire_gather(h, p):
        for lg in range(_BLK // _L):
            v = idx_v[h, pl.ds(lg * _L, _L)]
            ridx[p, pl.ds(lg * _L, _L)] = lax.shift_right_logical(v, 1)
        pltpu.async_copy(table_hbm.at[ridx.at[p]], rbuf.at[p], gsem[p])

    def extract(h, p):
        for lg in range(_BLK // _L):
            v = idx_v[h, pl.ds(lg * _L, _L)]
            base = (v & 1) * _D
            for k in range(_D):
                vals = plsc.load_gather(rbuf.at[p], [lvecs[lg], base + k])
                tbuf[p, k, pl.ds(lg * _L, _L)] = vals

    def fire_write(h, p):
        pltpu.async_copy(
            tbuf.at[p], out_hbm.at[h].at[:, pl.ds(b0, _BLK)], wsem[p]
        )

    def drain_g(p):
        pltpu.make_async_copy(
            table_hbm.at[pl.ds(0, _BLK)], rbuf.at[p], gsem[p]
        ).wait()

    def drain_w(p):
        pltpu.make_async_copy(
            out_hbm.at[0].at[:, pl.ds(0, _BLK)], tbuf.at[p], wsem[p]
        ).wait()

    prep_and_fire_gather(0, 0)
    prep_and_fire_gather(1, 1)

    def step(h2, carry):
        for p in range(2):
            h = h2 * 2 + p

            @pl.when(h >= 2)
            def _():
                drain_w(p)

            drain_g(p)
            extract(h, p)
            fire_write(h, p)

            @pl.when(h + 2 < hist)
            def _():
                prep_and_fire_gather(h + 2, p)

        return carry

    lax.fori_loop(0, hist // 2, step, 0)
    drain_w(0)
    drain_w(1)


def kernel(inputs, embeddings):
    b, h = inputs.shape
    v, d = embeddings.shape
    assert d == _D and b % (_NW * _BLK // _NW) == 0 and v % 2 == 0
    idx_t = jnp.transpose(inputs.astype(jnp.int32))
    table2 = jnp.reshape(embeddings, (v // 2, 2 * d))
    mesh = plsc.VectorSubcoreMesh(core_axis_name="c", subcore_axis_name="s")
    fn = pl.kernel(
        _gather_body,
        mesh=mesh,
        out_type=jax.ShapeDtypeStruct((h, d, b), jnp.float32),
        scratch_types=[
            pltpu.VMEM((h, _BLK), jnp.int32),
            pltpu.VMEM((2, _BLK), jnp.int32),
            pltpu.VMEM((2, _BLK, 2 * d), jnp.float32),
            pltpu.VMEM((2, d, _BLK), jnp.float32),
            pltpu.SemaphoreType.DMA,
            pltpu.SemaphoreType.DMA,
            pltpu.SemaphoreType.DMA,
            pltpu.SemaphoreType.DMA,
        ],
        compiler_params=pltpu.CompilerParams(needs_layout_passes=False),
    )
    out_t = fn(idx_t, table2)
    return jnp.transpose(out_t, (2, 0, 1))


# speedup vs baseline: 1.4027x; 1.4027x over previous
import jax
import jax.numpy as jnp
from jax import lax
from jax.experimental import pallas as pl
from jax.experimental.pallas import tpu as pltpu
from jax.experimental.pallas import tpu_sc as plsc

_D = 64
_NC = 2
_NS = 16
_NW = _NC * _NS
_L = 16
_BLK = 128


def _gather_body(idx_hbm, table_hbm, out_hbm, idx_v, ridx, rbuf, tbuf,
                 g0, g1, w0, w1):
    gsem = (g0, g1)
    wsem = (w0, w1)
    wid = lax.axis_index("s") * _NC + lax.axis_index("c")
    hist = idx_hbm.shape[0]
    b0 = wid * _BLK
    pltpu.sync_copy(idx_hbm.at[:, pl.ds(b0, _BLK)], idx_v)

    lane = lax.iota(jnp.int32, _L)
    lvecs = [lane + lg * _L for lg in range(_BLK // _L)]

    def prep_and_fire_gather(h, p):
        for lg in range(_BLK // _L):
            v = idx_v[h, pl.ds(lg * _L, _L)]
            ridx[p, pl.ds(lg * _L, _L)] = lax.shift_right_logical(v, 1)
        pltpu.async_copy(table_hbm.at[ridx.at[p]], rbuf.at[p], gsem[p])

    def extract(h, p):
        @plsc.parallel_loop(0, _BLK, step=_L)
        def _(l0):
            lvec = lane + l0
            v = idx_v[h, pl.ds(l0, _L)]
            base = (v & 1) * _D
            for k in range(_D):
                vals = plsc.load_gather(rbuf.at[p], [lvec, base + k])
                tbuf[p, k, pl.ds(l0, _L)] = vals

    def fire_write(h, p):
        pltpu.async_copy(
            tbuf.at[p], out_hbm.at[h].at[:, pl.ds(b0, _BLK)], wsem[p]
        )

    def drain_g(p):
        pltpu.make_async_copy(
            table_hbm.at[pl.ds(0, _BLK)], rbuf.at[p], gsem[p]
        ).wait()

    def drain_w(p):
        pltpu.make_async_copy(
            out_hbm.at[0].at[:, pl.ds(0, _BLK)], tbuf.at[p], wsem[p]
        ).wait()

    prep_and_fire_gather(0, 0)
    prep_and_fire_gather(1, 1)

    def step(h2, carry):
        for p in range(2):
            h = h2 * 2 + p

            @pl.when(h >= 2)
            def _():
                drain_w(p)

            drain_g(p)
            extract(h, p)
            fire_write(h, p)

            @pl.when(h + 2 < hist)
            def _():
                prep_and_fire_gather(h + 2, p)

        return carry

    lax.fori_loop(0, hist // 2, step, 0)
    drain_w(0)
    drain_w(1)


def kernel(inputs, embeddings):
    b, h = inputs.shape
    v, d = embeddings.shape
    assert d == _D and b % (_NW * _BLK // _NW) == 0 and v % 2 == 0
    idx_t = jnp.transpose(inputs.astype(jnp.int32))
    table2 = jnp.reshape(embeddings, (v // 2, 2 * d))
    mesh = plsc.VectorSubcoreMesh(core_axis_name="c", subcore_axis_name="s")
    fn = pl.kernel(
        _gather_body,
        mesh=mesh,
        out_type=jax.ShapeDtypeStruct((h, d, b), jnp.float32),
        scratch_types=[
            pltpu.VMEM((h, _BLK), jnp.int32),
            pltpu.VMEM((2, _BLK), jnp.int32),
            pltpu.VMEM((2, _BLK, 2 * d), jnp.float32),
            pltpu.VMEM((2, d, _BLK), jnp.float32),
            pltpu.SemaphoreType.DMA,
            pltpu.SemaphoreType.DMA,
            pltpu.SemaphoreType.DMA,
            pltpu.SemaphoreType.DMA,
        ],
        compiler_params=pltpu.CompilerParams(needs_layout_passes=False),
    )
    out_t = fn(idx_t, table2)
    return jnp.transpose(out_t, (2, 0, 1))


# speedup vs baseline: 1.4341x; 1.0224x over previous
import jax
import jax.numpy as jnp
from jax import lax
from jax.experimental import pallas as pl
from jax.experimental.pallas import tpu as pltpu
from jax.experimental.pallas import tpu_sc as plsc

_D = 64
_NC = 2
_NS = 16
_NW = _NC * _NS
_L = 16
_BLK = 128
_NG = 4
_NT = 2


def _gather_body(idx_hbm, table_hbm, out_hbm, idx_v, ridx, rbuf, tbuf,
                 g0, g1, g2, g3, w0, w1):
    gsem = (g0, g1, g2, g3)
    wsem = (w0, w1)
    wid = lax.axis_index("s") * _NC + lax.axis_index("c")
    hist = idx_hbm.shape[0]
    b0 = wid * _BLK
    pltpu.sync_copy(idx_hbm.at[:, pl.ds(b0, _BLK)], idx_v)

    lane = lax.iota(jnp.int32, _L)

    def prep_and_fire_gather(h, p):
        for lg in range(_BLK // _L):
            v = idx_v[h, pl.ds(lg * _L, _L)]
            ridx[p, pl.ds(lg * _L, _L)] = lax.shift_right_logical(v, 1)
        pltpu.async_copy(table_hbm.at[ridx.at[p]], rbuf.at[p], gsem[p])

    def extract(h, p, q):
        @plsc.parallel_loop(0, _BLK, step=_L)
        def _(l0):
            lvec = lane + l0
            v = idx_v[h, pl.ds(l0, _L)]
            base = (v & 1) * _D
            for k in range(_D):
                vals = plsc.load_gather(rbuf.at[p], [lvec, base + k])
                tbuf[q, k, pl.ds(l0, _L)] = vals

    def fire_write(h, q):
        pltpu.async_copy(
            tbuf.at[q], out_hbm.at[h].at[:, pl.ds(b0, _BLK)], wsem[q]
        )

    def drain_g(p):
        pltpu.make_async_copy(
            table_hbm.at[pl.ds(0, _BLK)], rbuf.at[p], gsem[p]
        ).wait()

    def drain_w(q):
        pltpu.make_async_copy(
            out_hbm.at[0].at[:, pl.ds(0, _BLK)], tbuf.at[q], wsem[q]
        ).wait()

    for p in range(_NG):
        prep_and_fire_gather(p, p)

    def step(h4, carry):
        for p in range(_NG):
            h = h4 * _NG + p
            q = p % _NT

            @pl.when(h >= _NT)
            def _():
                drain_w(q)

            drain_g(p)
            extract(h, p, q)
            fire_write(h, q)

            @pl.when(h + _NG < hist)
            def _():
                prep_and_fire_gather(h + _NG, p)

        return carry

    lax.fori_loop(0, hist // _NG, step, 0)
    drain_w(0)
    drain_w(1)


def kernel(inputs, embeddings):
    b, h = inputs.shape
    v, d = embeddings.shape
    assert d == _D and b == _NW * _BLK and v % 2 == 0 and h % _NG == 0
    idx_t = jnp.transpose(inputs.astype(jnp.int32))
    table2 = jnp.reshape(embeddings, (v // 2, 2 * d))
    mesh = plsc.VectorSubcoreMesh(core_axis_name="c", subcore_axis_name="s")
    fn = pl.kernel(
        _gather_body,
        mesh=mesh,
        out_type=jax.ShapeDtypeStruct((h, d, b), jnp.float32),
        scratch_types=[
            pltpu.VMEM((h, _BLK), jnp.int32),
            pltpu.VMEM((_NG, _BLK), jnp.int32),
            pltpu.VMEM((_NG, _BLK, 2 * d), jnp.float32),
            pltpu.VMEM((_NT, d, _BLK), jnp.float32),
            pltpu.SemaphoreType.DMA,
            pltpu.SemaphoreType.DMA,
            pltpu.SemaphoreType.DMA,
            pltpu.SemaphoreType.DMA,
            pltpu.SemaphoreType.DMA,
            pltpu.SemaphoreType.DMA,
        ],
        compiler_params=pltpu.CompilerParams(needs_layout_passes=False),
    )
    out_t = fn(idx_t, table2)
    return jnp.transpose(out_t, (2, 0, 1))


# speedup vs baseline: 1.9345x; 1.3489x over previous
import jax
import jax.numpy as jnp
from jax import lax
from jax.experimental import pallas as pl
from jax.experimental.pallas import tpu as pltpu
from jax.experimental.pallas import tpu_sc as plsc

_D = 64
_NC = 2
_NS = 16
_NW = _NC * _NS
_L = 16
_BLK = 128
_NG = 4
_NT = 2


def _gather_body(idx_hbm, table_hbm, out_hbm, idx_v, rbuf, tbuf,
                 g0, g1, g2, g3, w0, w1):
    gsem = (g0, g1, g2, g3)
    wsem = (w0, w1)
    wid = lax.axis_index("s") * _NC + lax.axis_index("c")
    hist = idx_hbm.shape[0]
    b0 = wid * _BLK
    pltpu.sync_copy(idx_hbm.at[:, pl.ds(b0, _BLK)], idx_v)

    lane = lax.iota(jnp.int32, _L)
    rots = [(lane + s) & (_L - 1) for s in range(_L)]

    def fire_gather(h, p):
        pltpu.async_copy(table_hbm.at[idx_v.at[h]], rbuf.at[p], gsem[p])

    def extract(p, q):
        @plsc.parallel_loop(0, _BLK, step=_L)
        def _(l0):
            rows = lane + l0
            for kb in range(_D // _L):
                for s in range(_L):
                    kvec = kb * _L + rots[s]
                    vals = plsc.load_gather(rbuf.at[p], [rows, kvec])
                    plsc.store_scatter(tbuf.at[q], [kvec, rows], vals)

    def fire_write(h, q):
        pltpu.async_copy(
            tbuf.at[q], out_hbm.at[h].at[:, pl.ds(b0, _BLK)], wsem[q]
        )

    def drain_g(p):
        pltpu.make_async_copy(
            table_hbm.at[pl.ds(0, _BLK)], rbuf.at[p], gsem[p]
        ).wait()

    def drain_w(q):
        pltpu.make_async_copy(
            out_hbm.at[0].at[:, pl.ds(0, _BLK)], tbuf.at[q], wsem[q]
        ).wait()

    for p in range(_NG):
        fire_gather(p, p)

    def step(h4, carry):
        for p in range(_NG):
            h = h4 * _NG + p
            q = p % _NT

            @pl.when(h >= _NT)
            def _():
                drain_w(q)

            drain_g(p)
            extract(p, q)
            fire_write(h, q)

            @pl.when(h + _NG < hist)
            def _():
                fire_gather(h + _NG, p)

        return carry

    lax.fori_loop(0, hist // _NG, step, 0)
    drain_w(0)
    drain_w(1)


def kernel(inputs, embeddings):
    b, h = inputs.shape
    v, d = embeddings.shape
    assert d == _D and b == _NW * _BLK and h % _NG == 0
    idx_t = jnp.transpose(inputs.astype(jnp.int32))
    table2 = jnp.pad(embeddings, ((0, 0), (0, d)))
    mesh = plsc.VectorSubcoreMesh(core_axis_name="c", subcore_axis_name="s")
    fn = pl.kernel(
        _gather_body,
        mesh=mesh,
        out_type=jax.ShapeDtypeStruct((h, d, b), jnp.float32),
        scratch_types=[
            pltpu.VMEM((h, _BLK), jnp.int32),
            pltpu.VMEM((_NG, _BLK, 2 * d), jnp.float32),
            pltpu.VMEM((_NT, d, _BLK), jnp.float32),
            pltpu.SemaphoreType.DMA,
            pltpu.SemaphoreType.DMA,
            pltpu.SemaphoreType.DMA,
            pltpu.SemaphoreType.DMA,
            pltpu.SemaphoreType.DMA,
            pltpu.SemaphoreType.DMA,
        ],
        compiler_params=pltpu.CompilerParams(needs_layout_passes=False),
    )
    out_t = fn(idx_t, table2)
    return jnp.transpose(out_t, (2, 0, 1))


# speedup vs baseline: 1.9432x; 1.0045x over previous
import jax
import jax.numpy as jnp
from jax import lax
from jax.experimental import pallas as pl
from jax.experimental.pallas import tpu as pltpu
from jax.experimental.pallas import tpu_sc as plsc

_D = 64
_NC = 2
_NS = 16
_NW = _NC * _NS
_L = 16
_BLK = 128
_NG = 4
_NT = 2


def _gather_body(idx_hbm, table_hbm, out_hbm, idx_v, rbuf, tbuf,
                 g0, g1, g2, g3, w0, w1):
    gsem = (g0, g1, g2, g3)
    wsem = (w0, w1)
    wid = lax.axis_index("s") * _NC + lax.axis_index("c")
    hist = idx_hbm.shape[0]
    b0 = wid * _BLK
    pltpu.sync_copy(idx_hbm.at[:, pl.ds(b0, _BLK)], idx_v)

    lane = lax.iota(jnp.int32, _L)
    rots = [(lane + s) & (_L - 1) for s in range(_L)]

    def fire_gather(h, p):
        pltpu.async_copy(table_hbm.at[idx_v.at[h]], rbuf.at[p], gsem[p])

    def extract(p, q):
        @plsc.parallel_loop(0, _BLK, step=_L)
        def _(l0):
            rows = lane + l0
            for kb in range(_D // _L):
                for s in range(_L):
                    kvec = kb * _L + rots[s]
                    vals = plsc.load_gather(rbuf.at[p], [rows, kvec])
                    plsc.store_scatter(tbuf.at[q], [kvec, rows], vals)

    def fire_write(h, q):
        pltpu.async_copy(
            tbuf.at[q], out_hbm.at[h].at[:, pl.ds(b0, _BLK)], wsem[q]
        )

    def drain_g(p):
        pltpu.make_async_copy(
            table_hbm.at[pl.ds(0, _BLK)], rbuf.at[p], gsem[p]
        ).wait()

    def drain_w(q):
        pltpu.make_async_copy(
            out_hbm.at[0].at[:, pl.ds(0, _BLK)], tbuf.at[q], wsem[q]
        ).wait()

    for p in range(_NG - 1):
        fire_gather(p, p)

    def step(h4, carry):
        for p in range(_NG):
            h = h4 * _NG + p
            q = p % _NT

            @pl.when(h >= _NT)
            def _():
                drain_w(q)

            drain_g(p)

            @pl.when(h + _NG - 1 < hist)
            def _():
                fire_gather(h + _NG - 1, (p + _NG - 1) % _NG)

            extract(p, q)
            fire_write(h, q)

        return carry

    lax.fori_loop(0, hist // _NG, step, 0)
    drain_w(0)
    drain_w(1)


def kernel(inputs, embeddings):
    b, h = inputs.shape
    v, d = embeddings.shape
    assert d == _D and b == _NW * _BLK and h % _NG == 0
    idx_t = jnp.transpose(inputs.astype(jnp.int32))
    table2 = jnp.pad(embeddings, ((0, 0), (0, d)))
    mesh = plsc.VectorSubcoreMesh(core_axis_name="c", subcore_axis_name="s")
    fn = pl.kernel(
        _gather_body,
        mesh=mesh,
        out_type=jax.ShapeDtypeStruct((h, d, b), jnp.float32),
        scratch_types=[
            pltpu.VMEM((h, _BLK), jnp.int32),
            pltpu.VMEM((_NG, _BLK, 2 * d), jnp.float32),
            pltpu.VMEM((_NT, d, _BLK), jnp.float32),
            pltpu.SemaphoreType.DMA,
            pltpu.SemaphoreType.DMA,
            pltpu.SemaphoreType.DMA,
            pltpu.SemaphoreType.DMA,
            pltpu.SemaphoreType.DMA,
            pltpu.SemaphoreType.DMA,
        ],
        compiler_params=pltpu.CompilerParams(needs_layout_passes=False),
    )
    out_t = fn(idx_t, table2)
    return jnp.transpose(out_t, (2, 0, 1))


# speedup vs baseline: 1.9980x; 1.0282x over previous
import jax
import jax.numpy as jnp
from jax import lax
from jax.experimental import pallas as pl
from jax.experimental.pallas import tpu as pltpu
from jax.experimental.pallas import tpu_sc as plsc

_D = 64
_NC = 2
_NS = 16
_NW = _NC * _NS
_L = 16
_BLK = 128
_NG = 4
_NT = 4


def _gather_body(idx_hbm, table_hbm, out_hbm, idx_v, rbuf, tbuf,
                 g0, g1, g2, g3, w0, w1, w2, w3):
    gsem = (g0, g1, g2, g3)
    wsem = (w0, w1, w2, w3)
    wid = lax.axis_index("s") * _NC + lax.axis_index("c")
    hist = idx_hbm.shape[0]
    b0 = wid * _BLK
    pltpu.sync_copy(idx_hbm.at[:, pl.ds(b0, _BLK)], idx_v)

    lane = lax.iota(jnp.int32, _L)
    rots = [(lane + s) & (_L - 1) for s in range(_L)]

    def fire_gather(h, p):
        pltpu.async_copy(table_hbm.at[idx_v.at[h]], rbuf.at[p], gsem[p])

    def extract(p, q):
        @plsc.parallel_loop(0, _BLK, step=_L)
        def _(l0):
            rows = lane + l0
            for kb in range(_D // _L):
                for s in range(_L):
                    kvec = kb * _L + rots[s]
                    vals = plsc.load_gather(rbuf.at[p], [rows, kvec])
                    plsc.store_scatter(tbuf.at[q], [kvec, rows], vals)

    def fire_write(h, q):
        pltpu.async_copy(
            tbuf.at[q], out_hbm.at[h].at[:, pl.ds(b0, _BLK)], wsem[q]
        )

    def drain_g(p):
        pltpu.make_async_copy(
            table_hbm.at[pl.ds(0, _BLK)], rbuf.at[p], gsem[p]
        ).wait()

    def drain_w(q):
        pltpu.make_async_copy(
            out_hbm.at[0].at[:, pl.ds(0, _BLK)], tbuf.at[q], wsem[q]
        ).wait()

    for p in range(_NG - 1):
        fire_gather(p, p)

    def step(h4, carry):
        for p in range(_NG):
            h = h4 * _NG + p
            q = p % _NT

            @pl.when(h >= _NT)
            def _():
                drain_w(q)

            drain_g(p)

            @pl.when(h + _NG - 1 < hist)
            def _():
                fire_gather(h + _NG - 1, (p + _NG - 1) % _NG)

            extract(p, q)
            fire_write(h, q)

        return carry

    lax.fori_loop(0, hist // _NG, step, 0)
    for q in range(_NT):
        drain_w(q)


def kernel(inputs, embeddings):
    b, h = inputs.shape
    v, d = embeddings.shape
    assert d == _D and b == _NW * _BLK and h % _NG == 0
    idx_t = jnp.transpose(inputs.astype(jnp.int32))
    table2 = jnp.pad(embeddings, ((0, 0), (0, d)))
    mesh = plsc.VectorSubcoreMesh(core_axis_name="c", subcore_axis_name="s")
    fn = pl.kernel(
        _gather_body,
        mesh=mesh,
        out_type=jax.ShapeDtypeStruct((h, d, b), jnp.float32),
        scratch_types=[
            pltpu.VMEM((h, _BLK), jnp.int32),
            pltpu.VMEM((_NG, _BLK, 2 * d), jnp.float32),
            pltpu.VMEM((_NT, d, _BLK), jnp.float32),
            pltpu.SemaphoreType.DMA,
            pltpu.SemaphoreType.DMA,
            pltpu.SemaphoreType.DMA,
            pltpu.SemaphoreType.DMA,
            pltpu.SemaphoreType.DMA,
            pltpu.SemaphoreType.DMA,
            pltpu.SemaphoreType.DMA,
            pltpu.SemaphoreType.DMA,
        ],
        compiler_params=pltpu.CompilerParams(needs_layout_passes=False),
    )
    out_t = fn(idx_t, table2)
    return jnp.transpose(out_t, (2, 0, 1))
